# baseline (speedup 1.0000x reference)
import functools

import jax
import jax.numpy as jnp
from jax import lax
from jax.experimental import pallas as pl
from jax.experimental.pallas import tpu as pltpu
from jax.experimental.pallas import tpu_sc as plsc

B = 128
K = 4096
DPC = 8
N = K * DPC

NW = 32
ROWS_PER_W = B // NW
HALF = N // 2
KHALF = K // 2


def _dk_body(x_hbm, out_hbm, x0, x1, o0, o1, sx0, sx1, so0, so1):
    wid = lax.axis_index("s") * 2 + lax.axis_index("c")
    xbufs = (x0, x1)
    obufs = (o0, o1)
    xsems = (sx0, sx1)
    osems = (so0, so1)

    def in_copy(r, b):
        row = wid * ROWS_PER_W + r
        return pltpu.async_copy(x_hbm.at[row], xbufs[b], xsems[b])

    def run_win(xbuf, obuf, h):
        @plsc.parallel_loop(0, KHALF // 16, unroll=4)
        def win_body(i):
            zeros = jnp.zeros((16,), jnp.float32)
            for v in range(8):
                obuf[pl.ds(i * 128 + v * 16, 16)] = zeros
            kloc = h * KHALF + i * 16 + lax.iota(jnp.int32, 16)
            base7 = kloc * 7
            m = plsc.load_gather(xbuf, [base7])
            w = jnp.zeros((16,), jnp.int32)
            for j in range(1, DPC):
                cj = plsc.load_gather(xbuf, [base7 + j])
                gt = cj > m
                m = jnp.where(gt, cj, m)
                w = jnp.where(gt, j, w)
            oidx = kloc * 8 + w
            vals = plsc.load_gather(xbuf, [oidx])
            plsc.store_scatter(obuf, [oidx - h * HALF], vals)

    in_handles = [None] * ROWS_PER_W
    out_handles = [None] * (2 * ROWS_PER_W)
    in_handles[0] = in_copy(0, 0)
    for r in range(ROWS_PER_W):
        xb = r % 2
        if r + 1 < ROWS_PER_W:
            in_handles[r + 1] = in_copy(r + 1, (r + 1) % 2)
        row = wid * ROWS_PER_W + r
        for h in range(2):
            ob = h
            ci = 2 * r + h
            if ci >= 2:
                out_handles[ci - 2].wait()
            if h == 0:
                in_handles[r].wait()
            run_win(xbufs[xb], obufs[ob], h)
            out_handles[ci] = pltpu.async_copy(
                obufs[ob], out_hbm.at[row, pl.ds(h * HALF, HALF)], osems[ob])
    out_handles[-2].wait()
    out_handles[-1].wait()


@jax.jit
def kernel(x):
    mesh = plsc.VectorSubcoreMesh(core_axis_name="c", subcore_axis_name="s")
    run = functools.partial(
        pl.kernel,
        mesh=mesh,
        out_type=jax.ShapeDtypeStruct((B, N), jnp.float32),
        compiler_params=pltpu.CompilerParams(needs_layout_passes=False),
        scratch_types=[
            pltpu.VMEM((N,), jnp.float32),
            pltpu.VMEM((N,), jnp.float32),
            pltpu.VMEM((HALF,), jnp.float32),
            pltpu.VMEM((HALF,), jnp.float32),
            pltpu.SemaphoreType.DMA,
            pltpu.SemaphoreType.DMA,
            pltpu.SemaphoreType.DMA,
            pltpu.SemaphoreType.DMA,
        ],
    )(_dk_body)
    return run(x)
